# baseline (speedup 1.0000x reference)
import jax
import jax.numpy as jnp
from jax import lax
from jax.experimental import pallas as pl
from jax.experimental.pallas import tpu as pltpu
from jax.experimental.pallas import tpu_sc as plsc

D_MODEL = 512
NC = 2
NS = 16
NW = NC * NS
SBLK = 64
S1 = 1792
NB1 = S1 // SBLK
TPAD = 256
NPW = TPAD // NW
NTP = 192
LC = D_MODEL // 16


def _make_sc_body(nx):
    def body(tab_h, i0_h, i1_h, i2_h, ope, tabl, iv, peb, sem):
        wid = lax.axis_index("s") * NC + lax.axis_index("c")
        base = S1 + wid * NPW
        ct = pltpu.async_copy(tab_h, tabl, sem)
        c0 = pltpu.async_copy(i0_h.at[pl.ds(base, NPW)],
                              iv.at[pl.ds(0, NPW)], sem)
        c1 = pltpu.async_copy(i1_h.at[pl.ds(base, NPW)],
                              iv.at[pl.ds(NPW, NPW)], sem)
        c2 = pltpu.async_copy(i2_h.at[pl.ds(base, NPW)],
                              iv.at[pl.ds(2 * NPW, NPW)], sem)
        ct.wait()
        c0.wait()
        c1.wait()
        c2.wait()

        def _tok(t, carry):
            a0 = iv[pl.ds(t, 1)][0] * D_MODEL
            a1 = (iv[pl.ds(NPW + t, 1)][0] + nx) * D_MODEL
            a2 = (iv[pl.ds(2 * NPW + t, 1)][0] + 2 * nx) * D_MODEL
            b = t * D_MODEL
            for j in range(LC):
                o = j * 16
                peb[pl.ds(b + o, 16)] = (tabl[pl.ds(a0 + o, 16)]
                                         + tabl[pl.ds(a1 + o, 16)]
                                         + tabl[pl.ds(a2 + o, 16)])
            return carry

        lax.fori_loop(0, NPW, _tok, 0)
        pltpu.sync_copy(peb, ope.at[pl.ds(wid * NPW * D_MODEL,
                                          NPW * D_MODEL)])

    return body


def _sc_pe(tab_flat, i0, i1, i2, nx):
    mesh = plsc.VectorSubcoreMesh(core_axis_name="c", subcore_axis_name="s")
    f = pl.kernel(
        _make_sc_body(nx),
        mesh=mesh,
        out_type=jax.ShapeDtypeStruct((TPAD * D_MODEL,), jnp.float32),
        scratch_types=[
            pltpu.VMEM((NTP * D_MODEL,), jnp.float32),
            pltpu.VMEM((3 * NPW,), jnp.int32),
            pltpu.VMEM((NPW * D_MODEL,), jnp.float32),
            pltpu.SemaphoreType.DMA,
        ],
    )
    return f(tab_flat, i0, i1, i2)


def _make_tc1_body(nx):
    def body(x_ref, idx_ref, tab_ref, o_ref):
        r = lax.broadcasted_iota(jnp.int32, (SBLK, NTP), 1)
        m = (((idx_ref[0, 0, :][:, None]) == r).astype(jnp.float32)
             + ((idx_ref[0, 1, :][:, None] + nx) == r).astype(jnp.float32)
             + ((idx_ref[0, 2, :][:, None] + 2 * nx) == r).astype(jnp.float32))
        pe = jnp.dot(m, tab_ref[...], preferred_element_type=jnp.float32,
                     precision=lax.Precision.HIGHEST)
        o_ref[...] = x_ref[...] + pe[:, None, :]

    return body


def _tc2_body(x_ref, pe_ref, o1_ref, o_ref):
    del o1_ref
    o_ref[...] = x_ref[...] + pe_ref[...][:, None, :]


def kernel(x, pos_x, pos_y, stab, token_to_x, token_to_y, token_to_stab):
    B, S, DM = x.shape
    nx = pos_x.shape[0]
    tpad = S1 + TPAD - S
    i0 = jnp.pad(token_to_x[:S].astype(jnp.int32), (0, tpad))
    i1 = jnp.pad(token_to_y[:S].astype(jnp.int32), (0, tpad))
    i2 = jnp.pad(token_to_stab[:S].astype(jnp.int32), (0, tpad))
    tab = jnp.concatenate([pos_x, pos_y, stab], axis=0)
    tabp = jnp.pad(tab, ((0, NTP - tab.shape[0]), (0, 0)))

    pe_tail = _sc_pe(tabp.reshape(-1), i0, i1, i2, nx).reshape(TPAD, DM)

    idx_head = jnp.stack(
        [i0[:S1].reshape(NB1, SBLK),
         i1[:S1].reshape(NB1, SBLK),
         i2[:S1].reshape(NB1, SBLK)], axis=1)
    xt = jnp.transpose(x, (1, 0, 2))

    o1 = pl.pallas_call(
        _make_tc1_body(nx),
        grid=(NB1,),
        in_specs=[
            pl.BlockSpec((SBLK, B, DM), lambda s: (s, 0, 0)),
            pl.BlockSpec((1, 3, SBLK), lambda s: (s, 0, 0)),
            pl.BlockSpec((NTP, DM), lambda s: (0, 0)),
        ],
        out_specs=pl.BlockSpec((SBLK, B, DM), lambda s: (s, 0, 0)),
        out_shape=jax.ShapeDtypeStruct((S, B, DM), x.dtype),
    )(xt, idx_head, tabp)

    nb2 = pl.cdiv(S - S1, SBLK)
    out_t = pl.pallas_call(
        _tc2_body,
        grid=(nb2,),
        in_specs=[
            pl.BlockSpec((SBLK, B, DM), lambda s: (s + NB1, 0, 0)),
            pl.BlockSpec((SBLK, DM), lambda s: (s, 0)),
            pl.BlockSpec(memory_space=pl.ANY),
        ],
        out_specs=pl.BlockSpec((SBLK, B, DM), lambda s: (s + NB1, 0, 0)),
        out_shape=jax.ShapeDtypeStruct((S, B, DM), x.dtype),
        input_output_aliases={2: 0},
    )(xt, pe_tail, o1)
    return jnp.transpose(out_t, (1, 0, 2))


# speedup vs baseline: 1.0033x; 1.0033x over previous
import jax
import jax.numpy as jnp
from jax import lax
from jax.experimental import pallas as pl
from jax.experimental.pallas import tpu as pltpu
from jax.experimental.pallas import tpu_sc as plsc

D_MODEL = 512
NC = 2
NS = 16
NW = NC * NS
SBLK = 64
S1 = 1536
NB1 = S1 // SBLK
TPAD = 512
NPW = TPAD // NW
NTP = 192
LC = D_MODEL // 16


def _make_sc_body(nx):
    def body(tab_h, i0_h, i1_h, i2_h, ope, tabl, iv, peb, sem):
        wid = lax.axis_index("s") * NC + lax.axis_index("c")
        base = S1 + wid * NPW
        ct = pltpu.async_copy(tab_h, tabl, sem)
        c0 = pltpu.async_copy(i0_h.at[pl.ds(base, NPW)],
                              iv.at[pl.ds(0, NPW)], sem)
        c1 = pltpu.async_copy(i1_h.at[pl.ds(base, NPW)],
                              iv.at[pl.ds(NPW, NPW)], sem)
        c2 = pltpu.async_copy(i2_h.at[pl.ds(base, NPW)],
                              iv.at[pl.ds(2 * NPW, NPW)], sem)
        ct.wait()
        c0.wait()
        c1.wait()
        c2.wait()

        def _tok(t, carry):
            a0 = iv[pl.ds(t, 1)][0] * D_MODEL
            a1 = (iv[pl.ds(NPW + t, 1)][0] + nx) * D_MODEL
            a2 = (iv[pl.ds(2 * NPW + t, 1)][0] + 2 * nx) * D_MODEL
            b = t * D_MODEL
            for j in range(LC):
                o = j * 16
                peb[pl.ds(b + o, 16)] = (tabl[pl.ds(a0 + o, 16)]
                                         + tabl[pl.ds(a1 + o, 16)]
                                         + tabl[pl.ds(a2 + o, 16)])
            return carry

        lax.fori_loop(0, NPW, _tok, 0)
        pltpu.sync_copy(peb, ope.at[pl.ds(wid * NPW * D_MODEL,
                                          NPW * D_MODEL)])

    return body


def _sc_pe(tab_flat, i0, i1, i2, nx):
    mesh = plsc.VectorSubcoreMesh(core_axis_name="c", subcore_axis_name="s")
    f = pl.kernel(
        _make_sc_body(nx),
        mesh=mesh,
        out_type=jax.ShapeDtypeStruct((TPAD * D_MODEL,), jnp.float32),
        scratch_types=[
            pltpu.VMEM((NTP * D_MODEL,), jnp.float32),
            pltpu.VMEM((3 * NPW,), jnp.int32),
            pltpu.VMEM((NPW * D_MODEL,), jnp.float32),
            pltpu.SemaphoreType.DMA,
        ],
    )
    return f(tab_flat, i0, i1, i2)


def _make_tc1_body(nx):
    def body(x_ref, idx_ref, tab_ref, o_ref):
        r = lax.broadcasted_iota(jnp.int32, (SBLK, NTP), 1)
        m = (((idx_ref[0, 0, :][:, None]) == r).astype(jnp.float32)
             + ((idx_ref[0, 1, :][:, None] + nx) == r).astype(jnp.float32)
             + ((idx_ref[0, 2, :][:, None] + 2 * nx) == r).astype(jnp.float32))
        pe = jnp.dot(m, tab_ref[...], preferred_element_type=jnp.float32,
                     precision=lax.Precision.HIGHEST)
        o_ref[...] = x_ref[...] + pe[:, None, :]

    return body


def _tc2_body(x_ref, pe_ref, o1_ref, o_ref):
    del o1_ref
    o_ref[...] = x_ref[...] + pe_ref[...][:, None, :]


def kernel(x, pos_x, pos_y, stab, token_to_x, token_to_y, token_to_stab):
    B, S, DM = x.shape
    nx = pos_x.shape[0]
    tpad = S1 + TPAD - S
    i0 = jnp.pad(token_to_x[:S].astype(jnp.int32), (0, tpad))
    i1 = jnp.pad(token_to_y[:S].astype(jnp.int32), (0, tpad))
    i2 = jnp.pad(token_to_stab[:S].astype(jnp.int32), (0, tpad))
    tab = jnp.concatenate([pos_x, pos_y, stab], axis=0)
    tabp = jnp.pad(tab, ((0, NTP - tab.shape[0]), (0, 0)))

    pe_tail = _sc_pe(tabp.reshape(-1), i0, i1, i2, nx).reshape(TPAD, DM)

    idx_head = jnp.stack(
        [i0[:S1].reshape(NB1, SBLK),
         i1[:S1].reshape(NB1, SBLK),
         i2[:S1].reshape(NB1, SBLK)], axis=1)
    xt = jnp.transpose(x, (1, 0, 2))

    o1 = pl.pallas_call(
        _make_tc1_body(nx),
        grid=(NB1,),
        in_specs=[
            pl.BlockSpec((SBLK, B, DM), lambda s: (s, 0, 0)),
            pl.BlockSpec((1, 3, SBLK), lambda s: (s, 0, 0)),
            pl.BlockSpec((NTP, DM), lambda s: (0, 0)),
        ],
        out_specs=pl.BlockSpec((SBLK, B, DM), lambda s: (s, 0, 0)),
        out_shape=jax.ShapeDtypeStruct((S, B, DM), x.dtype),
    )(xt, idx_head, tabp)

    nb2 = pl.cdiv(S - S1, SBLK)
    out_t = pl.pallas_call(
        _tc2_body,
        grid=(nb2,),
        in_specs=[
            pl.BlockSpec((SBLK, B, DM), lambda s: (s + NB1, 0, 0)),
            pl.BlockSpec((SBLK, DM), lambda s: (s, 0)),
            pl.BlockSpec(memory_space=pl.ANY),
        ],
        out_specs=pl.BlockSpec((SBLK, B, DM), lambda s: (s + NB1, 0, 0)),
        out_shape=jax.ShapeDtypeStruct((S, B, DM), x.dtype),
        input_output_aliases={2: 0},
    )(xt, pe_tail, o1)
    return jnp.transpose(out_t, (1, 0, 2))
